# baseline (speedup 1.0000x reference)
import functools

import jax
import jax.numpy as jnp
from jax import lax
from jax.experimental import pallas as pl
from jax.experimental.pallas import tpu as pltpu
from jax.experimental.pallas import tpu_sc as plsc

_CHUNK = 128
_LANES = 16


def kernel(labels, embedding_table, train):
    del train
    B = labels.shape[0]
    V, D = embedding_table.shape

    info = plsc.get_sparse_core_info()
    NC, NS = info.num_cores, info.num_subcores
    d_per_c = D // NC
    b_per_s = B // NS
    n_chunks = b_per_s // _CHUNK

    v_main = (V // 128) * 128
    n_tiles = v_main // 128
    tq, tr = divmod(n_tiles, NS)
    big = (tq + 1) * 128
    small = tq * 128

    tail = embedding_table[v_main:].T
    tail = jnp.pad(tail, ((0, 0), (0, 128 - tail.shape[1])))

    mesh = plsc.VectorSubcoreMesh(core_axis_name="c", subcore_axis_name="s")

    @functools.partial(
        pl.kernel,
        mesh=mesh,
        out_type=jax.ShapeDtypeStruct((D, B), jnp.float32),
        scratch_types=[
            pltpu.VMEM((b_per_s,), jnp.int32),
            pltpu.VMEM((b_per_s,), jnp.int32),
            pltpu.VMEM((D, 128), jnp.float32),
            pltpu.VMEM((d_per_c * b_per_s,), jnp.float32),
            pltpu.VMEM_SHARED((v_main,), jnp.float32),
            pltpu.SemaphoreType.DMA,
            pltpu.SemaphoreType.DMA,
            pltpu.SemaphoreType.DMA,
        ],
        compiler_params=pltpu.CompilerParams(needs_layout_passes=False),
    )
    def emb(idx_hbm, tab_hbm, tail_hbm, out_hbm, idx_v, idxc_v, tail_v,
            out_v, row0, row_sem, g_sem, w_sem):
        cid = lax.axis_index("c")
        sid = lax.axis_index("s")
        ibase = pl.multiple_of(sid * b_per_s, b_per_s)
        d0 = cid * d_per_c

        pltpu.sync_copy(idx_hbm.at[pl.ds(ibase, b_per_s)], idx_v)
        pltpu.sync_copy(tail_hbm, tail_v)

        mx = idx_v[pl.ds(0, _LANES)]
        for m in range(1, b_per_s // _LANES):
            v16 = idx_v[pl.ds(m * _LANES, _LANES)]
            mx = jnp.maximum(mx, v16)
        has_tail = plsc.all_reduce_population_count(mx >= v_main)[0] > 0
        for m in range(b_per_s // _LANES):
            v16 = idx_v[pl.ds(m * _LANES, _LANES)]
            idxc_v[pl.ds(m * _LANES, _LANES)] = jnp.minimum(v16, v_main - 1)

        sb = pl.multiple_of(
            jnp.minimum(sid, tr) * big
            + jnp.maximum(sid - tr, 0) * small,
            128,
        )

        def stage_row(d):
            @pl.when(sid < tr)
            def _():
                pltpu.async_copy(
                    tab_hbm.at[d].at[pl.ds(sb, big)],
                    row0.at[pl.ds(sb, big)],
                    row_sem,
                )

            @pl.when(sid >= tr)
            def _():
                pltpu.async_copy(
                    tab_hbm.at[d].at[pl.ds(sb, small)],
                    row0.at[pl.ds(sb, small)],
                    row_sem,
                )

        def wait_row():
            @pl.when(sid < tr)
            def _():
                pltpu.make_async_copy(
                    tab_hbm.at[0].at[pl.ds(0, big)],
                    row0.at[pl.ds(0, big)],
                    row_sem,
                ).wait()

            @pl.when(sid >= tr)
            def _():
                pltpu.make_async_copy(
                    tab_hbm.at[0].at[pl.ds(0, small)],
                    row0.at[pl.ds(0, small)],
                    row_sem,
                ).wait()

        stage_row(d0)

        def do_feature(k):
            wait_row()
            plsc.subcore_barrier()

            obase = pl.multiple_of(k * b_per_s, b_per_s)
            copies = []
            for m in range(n_chunks):
                copies.append(
                    pltpu.async_copy(
                        row0.at[idxc_v.at[pl.ds(m * _CHUNK, _CHUNK)]],
                        out_v.at[pl.ds(obase + m * _CHUNK, _CHUNK)],
                        g_sem,
                    )
                )
            for c in copies:
                c.wait()

            @pl.when(has_tail)
            def _():
                feat16 = jnp.broadcast_to(d0 + k, (_LANES,)).astype(jnp.int32)
                for m in range(b_per_s // _LANES):
                    o = obase + m * _LANES
                    i16 = idx_v[pl.ds(m * _LANES, _LANES)]
                    mask = i16 >= v_main
                    toff = jnp.maximum(i16 - v_main, 0)
                    tv = plsc.load_gather(tail_v, [feat16, toff])
                    cur = out_v[pl.ds(o, _LANES)]
                    out_v[pl.ds(o, _LANES)] = jnp.where(mask, tv, cur)

            pltpu.async_copy(
                out_v.at[pl.ds(obase, b_per_s)],
                out_hbm.at[d0 + k, pl.ds(ibase, b_per_s)],
                w_sem,
            )
            plsc.subcore_barrier()

            @pl.when(k + 1 < d_per_c)
            def _():
                stage_row(d0 + k + 1)

        pl.loop(0, d_per_c)(do_feature)

        pltpu.make_async_copy(
            tab_hbm.at[0].at[pl.ds(0, d_per_c * b_per_s)], out_v, w_sem
        ).wait()

    return emb(labels, embedding_table.T, tail).T
